# initial kernel scaffold (unmeasured)
import jax
import jax.numpy as jnp
from jax import lax
from jax.experimental import pallas as pl
from jax.experimental.pallas import tpu as pltpu


def kernel(
    x,
):
    def body(*refs):
        pass

    out_shape = jax.ShapeDtypeStruct(..., jnp.float32)
    return pl.pallas_call(body, out_shape=out_shape)(...)


# baseline (device time: 977144 ns/iter reference)
import jax
import jax.numpy as jnp
from jax import lax
from jax.experimental import pallas as pl
from jax.experimental.pallas import tpu as pltpu

M = 32768
N = 1024
CHUNK = 2048
NCH = M // CHUNK


def kernel(x):
    def body(
        x_ref,
        out_ref,
        send_buf,
        recv_buf,
        vx,
        vb,
        vo,
        local_sems,
        send_sem,
        recv_sem,
    ):
        mx = lax.axis_index("x")
        my = lax.axis_index("y")
        mz = lax.axis_index("z")
        mirror = (1 - mx, my, mz)

        barrier = pltpu.get_barrier_semaphore()
        pl.semaphore_signal(
            barrier, inc=1, device_id=mirror, device_id_type=pl.DeviceIdType.MESH
        )
        pl.semaphore_wait(barrier, 1)

        def conv(c, _):
            cp_in = pltpu.make_async_copy(
                x_ref.at[pl.ds(c * CHUNK, CHUNK), :], vx, local_sems.at[0]
            )
            cp_in.start()
            cp_in.wait()
            vb[...] = vx[...].astype(jnp.bfloat16)
            cp_out = pltpu.make_async_copy(
                vb, send_buf.at[pl.ds(c * CHUNK, CHUNK), :], local_sems.at[1]
            )
            cp_out.start()
            cp_out.wait()
            return 0

        lax.fori_loop(0, NCH, conv, 0)

        rdma = pltpu.make_async_remote_copy(
            src_ref=send_buf,
            dst_ref=recv_buf,
            send_sem=send_sem,
            recv_sem=recv_sem,
            device_id=mirror,
            device_id_type=pl.DeviceIdType.MESH,
        )
        rdma.start()
        rdma.wait()

        def addl(c, _):
            cp_x = pltpu.make_async_copy(
                x_ref.at[pl.ds(c * CHUNK, CHUNK), :], vx, local_sems.at[0]
            )
            cp_x.start()
            cp_r = pltpu.make_async_copy(
                recv_buf.at[pl.ds(c * CHUNK, CHUNK), :], vb, local_sems.at[1]
            )
            cp_r.start()
            cp_x.wait()
            cp_r.wait()
            vo[...] = (vx[...] + vb[...].astype(jnp.float32)).astype(jnp.bfloat16)
            cp_o = pltpu.make_async_copy(
                vo, out_ref.at[pl.ds(c * CHUNK, CHUNK), :], local_sems.at[2]
            )
            cp_o.start()
            cp_o.wait()
            return 0

        lax.fori_loop(0, NCH, addl, 0)

    out, _send, _recv = pl.pallas_call(
        body,
        out_shape=(
            jax.ShapeDtypeStruct((M, N), jnp.bfloat16),
            jax.ShapeDtypeStruct((M, N), jnp.bfloat16),
            jax.ShapeDtypeStruct((M, N), jnp.bfloat16),
        ),
        in_specs=[pl.BlockSpec(memory_space=pl.ANY)],
        out_specs=(
            pl.BlockSpec(memory_space=pl.ANY),
            pl.BlockSpec(memory_space=pl.ANY),
            pl.BlockSpec(memory_space=pl.ANY),
        ),
        scratch_shapes=[
            pltpu.VMEM((CHUNK, N), jnp.float32),
            pltpu.VMEM((CHUNK, N), jnp.bfloat16),
            pltpu.VMEM((CHUNK, N), jnp.bfloat16),
            pltpu.SemaphoreType.DMA((3,)),
            pltpu.SemaphoreType.DMA,
            pltpu.SemaphoreType.DMA,
        ],
        compiler_params=pltpu.CompilerParams(collective_id=0),
    )(x)
    return out


# device time: 346366 ns/iter; 2.8211x vs baseline; 2.8211x over previous
import jax
import jax.numpy as jnp
from jax import lax
from jax.experimental import pallas as pl
from jax.experimental.pallas import tpu as pltpu

M = 32768
N = 1024
Q = M // 4
NC = 8
CH = Q // NC
H = CH // 2

_MESH = pl.DeviceIdType.MESH


def kernel(x):
    def body(
        x_ref,
        out_ref,
        xsend_q,
        xrecv_q,
        s_q,
        vx,
        load_sem,
        out_sems,
        xsend_sems,
        xrecv_sems,
        zsend_sems,
        zrecv_sems,
        ysend_sems,
        yrecv_sems,
        zfsend_sems,
        zfrecv_sems,
        yfsend_sems,
        yfrecv_sems,
    ):
        mx = lax.axis_index("x")
        my = lax.axis_index("y")
        mz = lax.axis_index("z")
        mirror = (1 - mx, my, mz)
        znbr = (mx, my, 1 - mz)
        ynbr = (mx, 1 - my, mz)

        q = 2 * my + mz
        qz = 2 * my + (1 - mz)
        qy = 2 * (1 - my) + mz
        qd = 2 * (1 - my) + (1 - mz)

        def rows(p, c, n=CH, off=0):
            return pl.ds(p * Q + c * CH + off, n)

        def x_rdma(c):
            return pltpu.make_async_remote_copy(
                src_ref=xsend_q.at[pl.ds(c * CH, CH), :],
                dst_ref=xrecv_q.at[pl.ds(c * CH, CH), :],
                send_sem=xsend_sems.at[c],
                recv_sem=xrecv_sems.at[c],
                device_id=mirror,
                device_id_type=_MESH,
            )

        def z_rdma(c):
            return pltpu.make_async_remote_copy(
                src_ref=s_q.at[pl.ds(c * CH, CH), :],
                dst_ref=out_ref.at[rows(q, c), :],
                send_sem=zsend_sems.at[c],
                recv_sem=zrecv_sems.at[c],
                device_id=znbr,
                device_id_type=_MESH,
            )

        def y_rdma(c):
            return pltpu.make_async_remote_copy(
                src_ref=s_q.at[pl.ds(c * CH, CH), :],
                dst_ref=out_ref.at[rows(q, c), :],
                send_sem=ysend_sems.at[c],
                recv_sem=yrecv_sems.at[c],
                device_id=ynbr,
                device_id_type=_MESH,
            )

        def zrecv_desc(c):
            return pltpu.make_async_remote_copy(
                src_ref=s_q.at[pl.ds(c * CH, CH), :],
                dst_ref=out_ref.at[rows(qz, c), :],
                send_sem=zsend_sems.at[c],
                recv_sem=zrecv_sems.at[c],
                device_id=znbr,
                device_id_type=_MESH,
            )

        def yrecv_desc(c):
            return pltpu.make_async_remote_copy(
                src_ref=s_q.at[pl.ds(c * CH, CH), :],
                dst_ref=out_ref.at[rows(qy, c), :],
                send_sem=ysend_sems.at[c],
                recv_sem=yrecv_sems.at[c],
                device_id=ynbr,
                device_id_type=_MESH,
            )

        def yfwd_rdma(c):
            return pltpu.make_async_remote_copy(
                src_ref=out_ref.at[rows(qz, c, H), :],
                dst_ref=out_ref.at[rows(qz, c, H), :],
                send_sem=yfsend_sems.at[c],
                recv_sem=yfrecv_sems.at[c],
                device_id=ynbr,
                device_id_type=_MESH,
            )

        def zfwd_rdma(c):
            return pltpu.make_async_remote_copy(
                src_ref=out_ref.at[rows(qy, c, H, H), :],
                dst_ref=out_ref.at[rows(qy, c, H, H), :],
                send_sem=zfsend_sems.at[c],
                recv_sem=zfrecv_sems.at[c],
                device_id=znbr,
                device_id_type=_MESH,
            )

        def yfwd_recv_desc(c):
            return pltpu.make_async_remote_copy(
                src_ref=out_ref.at[rows(qd, c, H), :],
                dst_ref=out_ref.at[rows(qd, c, H), :],
                send_sem=yfsend_sems.at[c],
                recv_sem=yfrecv_sems.at[c],
                device_id=ynbr,
                device_id_type=_MESH,
            )

        def zfwd_recv_desc(c):
            return pltpu.make_async_remote_copy(
                src_ref=out_ref.at[rows(qd, c, H, H), :],
                dst_ref=out_ref.at[rows(qd, c, H, H), :],
                send_sem=zfsend_sems.at[c],
                recv_sem=zfrecv_sems.at[c],
                device_id=znbr,
                device_id_type=_MESH,
            )

        barrier = pltpu.get_barrier_semaphore()
        for nbr in (mirror, znbr, ynbr):
            pl.semaphore_signal(barrier, inc=1, device_id=nbr, device_id_type=_MESH)
        pl.semaphore_wait(barrier, 3)

        for c in range(NC):
            cp = pltpu.make_async_copy(x_ref.at[rows(q, c), :], vx, load_sem)
            cp.start()
            cp.wait()
            xsend_q[pl.ds(c * CH, CH), :] = vx[...].astype(jnp.bfloat16)
            x_rdma(c).start()

        for c in range(NC):
            x_rdma(c).wait_recv()
            s_q[pl.ds(c * CH, CH), :] = (
                xsend_q[pl.ds(c * CH, CH), :].astype(jnp.float32)
                + xrecv_q[pl.ds(c * CH, CH), :].astype(jnp.float32)
            ).astype(jnp.bfloat16)
            pltpu.make_async_copy(
                s_q.at[pl.ds(c * CH, CH), :], out_ref.at[rows(q, c), :], out_sems.at[c]
            ).start()
            z_rdma(c).start()
            y_rdma(c).start()

        for c in range(NC):
            zrecv_desc(c).wait_recv()
            yfwd_rdma(c).start()
            yrecv_desc(c).wait_recv()
            zfwd_rdma(c).start()

        for c in range(NC):
            yfwd_recv_desc(c).wait_recv()
            zfwd_recv_desc(c).wait_recv()

        for c in range(NC):
            x_rdma(c).wait_send()
            z_rdma(c).wait_send()
            y_rdma(c).wait_send()
            yfwd_rdma(c).wait_send()
            zfwd_rdma(c).wait_send()
            pltpu.make_async_copy(
                s_q.at[pl.ds(c * CH, CH), :], out_ref.at[rows(q, c), :], out_sems.at[c]
            ).wait()

    out = pl.pallas_call(
        body,
        out_shape=jax.ShapeDtypeStruct((M, N), jnp.bfloat16),
        in_specs=[pl.BlockSpec(memory_space=pl.ANY)],
        out_specs=pl.BlockSpec(memory_space=pl.ANY),
        scratch_shapes=[
            pltpu.VMEM((Q, N), jnp.bfloat16),
            pltpu.VMEM((Q, N), jnp.bfloat16),
            pltpu.VMEM((Q, N), jnp.bfloat16),
            pltpu.VMEM((CH, N), jnp.float32),
            pltpu.SemaphoreType.DMA,
            pltpu.SemaphoreType.DMA((NC,)),
            pltpu.SemaphoreType.DMA((NC,)),
            pltpu.SemaphoreType.DMA((NC,)),
            pltpu.SemaphoreType.DMA((NC,)),
            pltpu.SemaphoreType.DMA((NC,)),
            pltpu.SemaphoreType.DMA((NC,)),
            pltpu.SemaphoreType.DMA((NC,)),
            pltpu.SemaphoreType.DMA((NC,)),
            pltpu.SemaphoreType.DMA((NC,)),
            pltpu.SemaphoreType.DMA((NC,)),
            pltpu.SemaphoreType.DMA((NC,)),
        ],
        compiler_params=pltpu.CompilerParams(
            collective_id=0, vmem_limit_bytes=96 * 1024 * 1024
        ),
    )(x)
    return out


# device time: 315115 ns/iter; 3.1009x vs baseline; 1.0992x over previous
import jax
import jax.numpy as jnp
from jax import lax
from jax.experimental import pallas as pl
from jax.experimental.pallas import tpu as pltpu

M = 32768
N = 1024
Q = M // 4
NC = 8
CH = Q // NC

ZF = [(0, 1024), (1024, 1024), (2048, 688)]
YF = [(2736, 336), (3072, 1024), (4096, 1024), (5120, 352)]
SELF = [(5472, 1024), (6496, 1024), (7520, 672)]
NS = len(SELF)
SELF0 = SELF[0][0]
YF_SRC = [o // CH for o, _ in YF]
ZF_SRC = [o // CH for o, _ in ZF]
XR = Q + sum(n for _, n in SELF)

_MESH = pl.DeviceIdType.MESH


def kernel(x):
    def body(
        x_ref,
        out_ref,
        xsend,
        xrecv,
        vx,
        load_sem,
        out_sems,
        xsend_sems,
        xrecv_sems,
        zsend_sems,
        zrecv_sems,
        ysend_sems,
        yrecv_sems,
        zfsend_sems,
        zfrecv_sems,
        yfsend_sems,
        yfrecv_sems,
    ):
        mx = lax.axis_index("x")
        my = lax.axis_index("y")
        mz = lax.axis_index("z")
        mirror = (1 - mx, my, mz)
        znbr = (mx, my, 1 - mz)
        ynbr = (mx, 1 - my, mz)

        q = 2 * my + mz
        qz = 2 * my + (1 - mz)
        qy = 2 * (1 - my) + mz
        qd = 2 * (1 - my) + (1 - mz)

        def rows(p, off, n):
            return pl.ds(p * Q + off, n)

        def x_rdma(c):
            return pltpu.make_async_remote_copy(
                src_ref=xsend.at[pl.ds(c * CH, CH), :],
                dst_ref=xrecv.at[pl.ds(c * CH, CH), :],
                send_sem=xsend_sems.at[c],
                recv_sem=xrecv_sems.at[c],
                device_id=mirror,
                device_id_type=_MESH,
            )

        def xs_rdma(i):
            boff = Q + (SELF[i][0] - SELF0)
            return pltpu.make_async_remote_copy(
                src_ref=xsend.at[pl.ds(boff, SELF[i][1]), :],
                dst_ref=xrecv.at[pl.ds(boff, SELF[i][1]), :],
                send_sem=xsend_sems.at[NC + i],
                recv_sem=xrecv_sems.at[NC + i],
                device_id=mirror,
                device_id_type=_MESH,
            )

        def z_rdma(c):
            return pltpu.make_async_remote_copy(
                src_ref=xsend.at[pl.ds(c * CH, CH), :],
                dst_ref=out_ref.at[rows(q, c * CH, CH), :],
                send_sem=zsend_sems.at[c],
                recv_sem=zrecv_sems.at[c],
                device_id=znbr,
                device_id_type=_MESH,
            )

        def y_rdma(c):
            return pltpu.make_async_remote_copy(
                src_ref=xsend.at[pl.ds(c * CH, CH), :],
                dst_ref=out_ref.at[rows(q, c * CH, CH), :],
                send_sem=ysend_sems.at[c],
                recv_sem=yrecv_sems.at[c],
                device_id=ynbr,
                device_id_type=_MESH,
            )

        def zrecv_desc(c):
            return pltpu.make_async_remote_copy(
                src_ref=xsend.at[pl.ds(c * CH, CH), :],
                dst_ref=out_ref.at[rows(qz, c * CH, CH), :],
                send_sem=zsend_sems.at[c],
                recv_sem=zrecv_sems.at[c],
                device_id=znbr,
                device_id_type=_MESH,
            )

        def yrecv_desc(c):
            return pltpu.make_async_remote_copy(
                src_ref=xsend.at[pl.ds(c * CH, CH), :],
                dst_ref=out_ref.at[rows(qy, c * CH, CH), :],
                send_sem=ysend_sems.at[c],
                recv_sem=yrecv_sems.at[c],
                device_id=ynbr,
                device_id_type=_MESH,
            )

        def zf_rdma(i):
            o, n = ZF[i]
            return pltpu.make_async_remote_copy(
                src_ref=out_ref.at[rows(qy, o, n), :],
                dst_ref=out_ref.at[rows(qy, o, n), :],
                send_sem=zfsend_sems.at[i],
                recv_sem=zfrecv_sems.at[i],
                device_id=znbr,
                device_id_type=_MESH,
            )

        def yf_rdma(i):
            o, n = YF[i]
            return pltpu.make_async_remote_copy(
                src_ref=out_ref.at[rows(qz, o, n), :],
                dst_ref=out_ref.at[rows(qz, o, n), :],
                send_sem=yfsend_sems.at[i],
                recv_sem=yfrecv_sems.at[i],
                device_id=ynbr,
                device_id_type=_MESH,
            )

        def zf_recv_desc(i):
            o, n = ZF[i]
            return pltpu.make_async_remote_copy(
                src_ref=out_ref.at[rows(qd, o, n), :],
                dst_ref=out_ref.at[rows(qd, o, n), :],
                send_sem=zfsend_sems.at[i],
                recv_sem=zfrecv_sems.at[i],
                device_id=znbr,
                device_id_type=_MESH,
            )

        def yf_recv_desc(i):
            o, n = YF[i]
            return pltpu.make_async_remote_copy(
                src_ref=out_ref.at[rows(qd, o, n), :],
                dst_ref=out_ref.at[rows(qd, o, n), :],
                send_sem=yfsend_sems.at[i],
                recv_sem=yfrecv_sems.at[i],
                device_id=ynbr,
                device_id_type=_MESH,
            )

        def out_dma(c):
            return pltpu.make_async_copy(
                xsend.at[pl.ds(c * CH, CH), :],
                out_ref.at[rows(q, c * CH, CH), :],
                out_sems.at[c],
            )

        def self_dma(i):
            o, n = SELF[i]
            boff = Q + (o - SELF0)
            return pltpu.make_async_copy(
                xsend.at[pl.ds(boff, n), :],
                out_ref.at[rows(qd, o, n), :],
                out_sems.at[NC + i],
            )

        barrier = pltpu.get_barrier_semaphore()
        for nbr in (mirror, znbr, ynbr):
            pl.semaphore_signal(barrier, inc=1, device_id=nbr, device_id_type=_MESH)
        pl.semaphore_wait(barrier, 3)

        for c in range(NC):
            cp = pltpu.make_async_copy(x_ref.at[rows(q, c * CH, CH), :], vx, load_sem)
            cp.start()
            cp.wait()
            xsend[pl.ds(c * CH, CH), :] = vx[...].astype(jnp.bfloat16)
            x_rdma(c).start()

        for i in range(NS):
            o, n = SELF[i]
            boff = Q + (o - SELF0)
            cp = pltpu.make_async_copy(
                x_ref.at[rows(qd, o, n), :], vx.at[pl.ds(0, n), :], load_sem
            )
            cp.start()
            cp.wait()
            xsend[pl.ds(boff, n), :] = vx[pl.ds(0, n), :].astype(jnp.bfloat16)
            xs_rdma(i).start()

        for c in range(NC):
            x_rdma(c).wait_recv()
            x_rdma(c).wait_send()
            xsend[pl.ds(c * CH, CH), :] = (
                xsend[pl.ds(c * CH, CH), :].astype(jnp.float32)
                + xrecv[pl.ds(c * CH, CH), :].astype(jnp.float32)
            ).astype(jnp.bfloat16)
            out_dma(c).start()
            z_rdma(c).start()
            y_rdma(c).start()

        for c in range(NC):
            zrecv_desc(c).wait_recv()
            for i in range(len(YF)):
                if YF_SRC[i] == c:
                    yf_rdma(i).start()
            yrecv_desc(c).wait_recv()
            for i in range(len(ZF)):
                if ZF_SRC[i] == c:
                    zf_rdma(i).start()

        for i in range(NS):
            o, n = SELF[i]
            boff = Q + (o - SELF0)
            xs_rdma(i).wait_recv()
            xs_rdma(i).wait_send()
            xsend[pl.ds(boff, n), :] = (
                xsend[pl.ds(boff, n), :].astype(jnp.float32)
                + xrecv[pl.ds(boff, n), :].astype(jnp.float32)
            ).astype(jnp.bfloat16)
            self_dma(i).start()

        for i in range(len(ZF)):
            zf_recv_desc(i).wait_recv()
        for i in range(len(YF)):
            yf_recv_desc(i).wait_recv()

        for c in range(NC):
            z_rdma(c).wait_send()
            y_rdma(c).wait_send()
            out_dma(c).wait()
        for i in range(NS):
            self_dma(i).wait()
        for i in range(len(ZF)):
            zf_rdma(i).wait_send()
        for i in range(len(YF)):
            yf_rdma(i).wait_send()

    out = pl.pallas_call(
        body,
        out_shape=jax.ShapeDtypeStruct((M, N), jnp.bfloat16),
        in_specs=[pl.BlockSpec(memory_space=pl.ANY)],
        out_specs=pl.BlockSpec(memory_space=pl.ANY),
        scratch_shapes=[
            pltpu.VMEM((XR, N), jnp.bfloat16),
            pltpu.VMEM((XR, N), jnp.bfloat16),
            pltpu.VMEM((CH, N), jnp.float32),
            pltpu.SemaphoreType.DMA,
            pltpu.SemaphoreType.DMA((NC + NS,)),
            pltpu.SemaphoreType.DMA((NC + NS,)),
            pltpu.SemaphoreType.DMA((NC + NS,)),
            pltpu.SemaphoreType.DMA((NC,)),
            pltpu.SemaphoreType.DMA((NC,)),
            pltpu.SemaphoreType.DMA((NC,)),
            pltpu.SemaphoreType.DMA((NC,)),
            pltpu.SemaphoreType.DMA((len(ZF),)),
            pltpu.SemaphoreType.DMA((len(ZF),)),
            pltpu.SemaphoreType.DMA((len(YF),)),
            pltpu.SemaphoreType.DMA((len(YF),)),
        ],
        compiler_params=pltpu.CompilerParams(
            collective_id=0, vmem_limit_bytes=96 * 1024 * 1024
        ),
    )(x)
    return out
